# baseline (speedup 1.0000x reference)
import functools

import jax
import jax.numpy as jnp
from jax import lax
from jax.experimental import pallas as pl
from jax.experimental.pallas import tpu as pltpu
from jax.experimental.pallas import tpu_sc as plsc

NUM_SAMPLES = 32
NUM_CORES = 2
NUM_SUBCORES = 16


def _subsample_rows(x2d, t, row):
    channels = x2d.shape[0] // t

    mesh = plsc.VectorSubcoreMesh(core_axis_name="c", subcore_axis_name="s")

    @functools.partial(
        pl.kernel,
        mesh=mesh,
        out_type=jax.ShapeDtypeStruct((channels * NUM_SAMPLES, row), jnp.float32),
        scratch_types=[
            pltpu.VMEM((row,), jnp.float32),
            pltpu.VMEM((row,), jnp.float32),
            pltpu.SemaphoreType.DMA,
            pltpu.SemaphoreType.DMA,
            pltpu.SemaphoreType.DMA,
            pltpu.SemaphoreType.DMA,
        ],
    )
    def k(x_hbm, out_hbm, buf0, buf1, si0, si1, so0, so1):
        w = lax.axis_index("s") * NUM_CORES + lax.axis_index("c")
        tsel = (w * (t - 1)) // (NUM_SAMPLES - 1)

        bufs = (buf0, buf1)
        isems = (si0, si1)
        osems = (so0, so1)

        h_out = [None, None]
        h_in = pltpu.async_copy(x_hbm.at[tsel], bufs[0], isems[0])
        for j in range(channels):
            nxt = (j + 1) % 2
            cur = j % 2
            if j + 1 < channels:
                if h_out[nxt] is not None:
                    h_out[nxt].wait()
                h_in_next = pltpu.async_copy(
                    x_hbm.at[(j + 1) * t + tsel], bufs[nxt], isems[nxt])
            h_in.wait()
            h_out[cur] = pltpu.async_copy(
                bufs[cur], out_hbm.at[j * NUM_SAMPLES + w], osems[cur])
            if j + 1 < channels:
                h_in = h_in_next
        for h in h_out:
            if h is not None:
                h.wait()

    return k(x2d)


def kernel(x):
    c, t, h, wdt = x.shape
    row = h * wdt
    x2d = x.reshape(c * t, row)
    out2d = _subsample_rows(x2d, t, row)
    return out2d.reshape(c, NUM_SAMPLES, h, wdt)


# speedup vs baseline: 1.8040x; 1.8040x over previous
import functools

import jax
import jax.numpy as jnp
from jax import lax
from jax.experimental import pallas as pl
from jax.experimental.pallas import tpu as pltpu
from jax.experimental.pallas import tpu_sc as plsc

NUM_SAMPLES = 32
NUM_CORES = 2
NUM_SUBCORES = 16


def kernel(x):
    channels, t, h, wdt = x.shape

    mesh = plsc.VectorSubcoreMesh(core_axis_name="c", subcore_axis_name="s")

    @functools.partial(
        pl.kernel,
        mesh=mesh,
        out_type=jax.ShapeDtypeStruct(
            (channels, NUM_SAMPLES, h, wdt), jnp.float32),
        scratch_types=[
            pltpu.VMEM((h, wdt), jnp.float32),
            pltpu.VMEM((h, wdt), jnp.float32),
            pltpu.SemaphoreType.DMA,
            pltpu.SemaphoreType.DMA,
            pltpu.SemaphoreType.DMA,
            pltpu.SemaphoreType.DMA,
        ],
    )
    def k(x_hbm, out_hbm, buf0, buf1, si0, si1, so0, so1):
        w = lax.axis_index("s") * NUM_CORES + lax.axis_index("c")
        tsel = (w * (t - 1)) // (NUM_SAMPLES - 1)

        bufs = (buf0, buf1)
        isems = (si0, si1)
        osems = (so0, so1)

        h_out = [None, None]
        h_in = pltpu.async_copy(x_hbm.at[0, tsel], bufs[0], isems[0])
        for j in range(channels):
            nxt = (j + 1) % 2
            cur = j % 2
            if j + 1 < channels:
                if h_out[nxt] is not None:
                    h_out[nxt].wait()
                h_in_next = pltpu.async_copy(
                    x_hbm.at[j + 1, tsel], bufs[nxt], isems[nxt])
            h_in.wait()
            h_out[cur] = pltpu.async_copy(
                bufs[cur], out_hbm.at[j, w], osems[cur])
            if j + 1 < channels:
                h_in = h_in_next
        for hd in h_out:
            if hd is not None:
                hd.wait()

    return k(x)
